# Initial kernel scaffold; baseline (speedup 1.0000x reference)
import jax
import jax.numpy as jnp
from jax.experimental import pallas as pl


def kernel(m, edge_index, W, b):
    raise NotImplementedError("write your pallas kernel here")


# speedup vs baseline: 2.0989x; 2.0989x over previous
import functools

import jax
import jax.numpy as jnp
from jax import lax
from jax.experimental import pallas as pl
from jax.experimental.pallas import tpu as pltpu
from jax.experimental.pallas import tpu_sc as plsc

N_NODES = 10000
N_EDGES = 320000
D = 128
NF = D // 16

NW = 32
NPT = 313
NPT_PAD = 320

C = 1280
NCHUNK = N_EDGES // C
NV = C // 16
K = 24


def _body(m_h, dst_h, wb_h, out_h,
          sum_v, min_v, max_v, deg_v, dst_v, ids_v, lds_v, rows_v,
          wb_v, sem_g):
    cid = lax.axis_index("c")
    sid = lax.axis_index("s")
    wid = sid * 2 + cid
    lo = wid * NPT
    hi = jnp.minimum(lo + NPT, N_NODES)

    pltpu.sync_copy(wb_h, wb_v)

    zero16f = jnp.zeros((16,), jnp.float32)
    pinf = jnp.full((16,), jnp.inf, jnp.float32)
    ninf = jnp.full((16,), -jnp.inf, jnp.float32)
    iota16 = lax.iota(jnp.int32, 16)

    def init_node(n, _):
        for f in range(NF):
            sl = pl.ds(f * 16, 16)
            sum_v[n, sl] = zero16f
            min_v[n, sl] = pinf
            max_v[n, sl] = ninf
        return 0
    lax.fori_loop(0, NPT, init_node, 0)

    zero16i = jnp.zeros((16,), jnp.int32)

    def init_deg(i, _):
        deg_v[i] = jnp.int32(0)
        return 0
    lax.fori_loop(0, 320, init_deg, 0)

    def init_ids(i, _):
        ids_v[pl.ds(i * 16, 16)] = zero16i
        return 0
    lax.fori_loop(0, (C + 16) // 16, init_ids, 0)

    def chunk_body(c, _):
        pltpu.sync_copy(dst_h.at[pl.ds(c * C, C)], dst_v)
        base_id = c * C + iota16

        def scan_body(v, mc):
            d = dst_v[pl.ds(v * 16, 16)]
            msk = (d >= lo) & (d < hi)
            key = jnp.where(msk, d - lo, jnp.int32(0x7FFFFFFF))
            skey, seid = plsc.sort_key_val(key, base_id + v * 16)
            lds_v[pl.ds(mc, 16)] = skey
            ids_v[pl.ds(mc, 16)] = seid
            cnt = plsc.all_reduce_population_count(msk)
            return mc + cnt[0]

        mc = lax.fori_loop(0, NV, scan_body, jnp.int32(0))

        ng = (mc + (K - 1)) // K

        def g_body(g, _):
            pltpu.async_copy(m_h.at[ids_v.at[pl.ds(g * K, K)]], rows_v,
                             sem_g).wait()
            kk = jnp.minimum(jnp.int32(K), mc - g * K)

            def e_body(j, _):
                ld = lds_v[pl.ds(g * K + j, 16)][0]
                for f in range(NF):
                    sl = pl.ds(f * 16, 16)
                    r = rows_v[j, sl]
                    sum_v[ld, sl] = sum_v[ld, sl] + r
                    min_v[ld, sl] = jnp.minimum(min_v[ld, sl], r)
                    max_v[ld, sl] = jnp.maximum(max_v[ld, sl], r)
                deg_v[ld] = deg_v[ld] + 1
                return 0

            lax.fori_loop(0, kk, e_body, 0)
            return 0

        lax.fori_loop(0, ng, g_body, 0)
        return 0

    lax.fori_loop(0, NCHUNK, chunk_body, 0)

    wbv = wb_v[pl.ds(0, 16)]
    w0 = jnp.broadcast_to(wbv[0], (16,))
    w1 = jnp.broadcast_to(wbv[1], (16,))
    w2 = jnp.broadcast_to(wbv[2], (16,))
    w3 = jnp.broadcast_to(wbv[3], (16,))
    bb = jnp.broadcast_to(wbv[4], (16,))

    def comb(n, _):
        dg = deg_v[n]
        df = dg.astype(jnp.float32)
        hasv = jnp.broadcast_to((dg > 0).astype(jnp.float32), (16,))
        dfv = jnp.broadcast_to(df, (16,))
        invd = jnp.float32(1.0) / jnp.maximum(dfv, jnp.float32(1.0))
        for f in range(NF):
            sl = pl.ds(f * 16, 16)
            s = sum_v[n, sl]
            mn = min_v[n, sl]
            mx = max_v[n, sl]
            val = w0 * s + hasv * (w1 * mn + w2 * mx) + w3 * (s * invd) + bb
            sum_v[n, sl] = val
        return 0

    lax.fori_loop(0, NPT, comb, 0)

    pltpu.sync_copy(sum_v, out_h.at[pl.ds(wid * NPT_PAD, NPT_PAD)])


@jax.jit
def _agg(m, dst, wb):
    mesh = plsc.VectorSubcoreMesh(core_axis_name="c", subcore_axis_name="s",
                                  num_cores=2, num_subcores=16)
    return pl.kernel(
        _body,
        out_type=jax.ShapeDtypeStruct((NW * NPT_PAD, D), jnp.float32),
        mesh=mesh,
        compiler_params=pltpu.CompilerParams(needs_layout_passes=False),
        scratch_types=[
            pltpu.VMEM((NPT_PAD, D), jnp.float32),
            pltpu.VMEM((NPT, D), jnp.float32),
            pltpu.VMEM((NPT, D), jnp.float32),
            pltpu.SMEM((320,), jnp.int32),
            pltpu.VMEM((C,), jnp.int32),
            pltpu.VMEM((C + 16,), jnp.int32),
            pltpu.VMEM((C + 16,), jnp.int32),
            pltpu.VMEM((K, D), jnp.float32),
            pltpu.VMEM((16,), jnp.float32),
            pltpu.SemaphoreType.DMA,
        ],
    )(m, dst, wb)


def kernel(m, edge_index, W, b):
    dst = edge_index[1]
    wb = jnp.concatenate([W.reshape(-1), b,
                          jnp.zeros((11,), jnp.float32)])
    out_pad = _agg(m, dst, wb)
    out = out_pad.reshape(NW, NPT_PAD, D)[:, :NPT].reshape(NW * NPT, D)
    return out[:N_NODES]


# speedup vs baseline: 2.2559x; 1.0748x over previous
import jax
import jax.numpy as jnp
from jax import lax
from jax.experimental import pallas as pl
from jax.experimental.pallas import tpu as pltpu
from jax.experimental.pallas import tpu_sc as plsc

N_NODES = 10000
N_EDGES = 320000
D = 128
NF = D // 16

NW = 32
NPT = 313
NPT_PAD = 320

C = 800
NCHUNK = N_EDGES // C
NV2 = C // 32
K = 16

def _body(m_h, dst_h, wb_h, out_h,
          sum_v, min_v, max_v, deg_v, dst0_v, dst1_v, ids_v, lds_v,
          rows0_v, rows1_v, wb_v, sem_d0, sem_d1, sem_g0, sem_g1):
    cid = lax.axis_index("c")
    sid = lax.axis_index("s")
    wid = sid * 2 + cid
    lo = wid * NPT
    hi = jnp.minimum(lo + NPT, N_NODES)

    pltpu.sync_copy(wb_h, wb_v)

    zero16f = jnp.zeros((16,), jnp.float32)
    pinf = jnp.full((16,), jnp.inf, jnp.float32)
    ninf = jnp.full((16,), -jnp.inf, jnp.float32)
    iota16 = lax.iota(jnp.int32, 16)

    def init_node(n, _):
        for f in range(NF):
            sl = pl.ds(f * 16, 16)
            sum_v[n, sl] = zero16f
            min_v[n, sl] = pinf
            max_v[n, sl] = ninf
        return 0
    lax.fori_loop(0, NPT, init_node, 0)

    zero16i = jnp.zeros((16,), jnp.int32)

    def init_deg(i, _):
        deg_v[i] = jnp.int32(0)
        return 0
    lax.fori_loop(0, 320, init_deg, 0)

    def init_ids(i, _):
        ids_v[pl.ds(i * 16, 16)] = zero16i
        return 0
    lax.fori_loop(0, (C + 16) // 16, init_ids, 0)

    def start_dst(c, dbuf, dsem):
        pltpu.async_copy(dst_h.at[pl.ds(c * C, C)], dbuf, dsem)

    def start_gather(g, rbuf, rsem):
        pltpu.async_copy(m_h.at[ids_v.at[pl.ds(g * K, K)]], rbuf, rsem)

    def wait_dst(sem, buf):
        pltpu.make_async_copy(dst_h.at[pl.ds(0, C)], buf, sem).wait()

    def wait_rows(sem, buf):
        pltpu.make_async_copy(m_h.at[pl.ds(0, K)], buf, sem).wait()

    def accum(rows_v, g, mc):
        kk = jnp.minimum(jnp.int32(K), mc - g * K)

        def e_body(j, _):
            ld = lds_v[pl.ds(g * K + j, 16)][0]
            for f in range(NF):
                sl = pl.ds(f * 16, 16)
                r = rows_v[j, sl]
                sum_v[ld, sl] = sum_v[ld, sl] + r
                min_v[ld, sl] = jnp.minimum(min_v[ld, sl], r)
                max_v[ld, sl] = jnp.maximum(max_v[ld, sl], r)
            deg_v[ld] = deg_v[ld] + 1
            return 0

        lax.fori_loop(0, kk, e_body, 0)

    def process_chunk(c, dst_v):
        base_id = c * C + iota16

        def scan_pair(v2, mc):
            off = v2 * 32
            d1 = dst_v[pl.ds(off, 16)]
            d2 = dst_v[pl.ds(off + 16, 16)]
            m1 = (d1 >= lo) & (d1 < hi)
            m2 = (d2 >= lo) & (d2 < hi)
            k1 = jnp.where(m1, d1 - lo, jnp.int32(0x7FFFFFFF))
            k2 = jnp.where(m2, d2 - lo, jnp.int32(0x7FFFFFFF))
            s1k, s1v = plsc.sort_key_val(k1, base_id + off)
            s2k, s2v = plsc.sort_key_val(k2, base_id + (off + 16))
            c1 = plsc.all_reduce_population_count(m1)[0]
            c2 = plsc.all_reduce_population_count(m2)[0]
            lds_v[pl.ds(mc, 16)] = s1k
            ids_v[pl.ds(mc, 16)] = s1v
            mcb = mc + c1
            lds_v[pl.ds(mcb, 16)] = s2k
            ids_v[pl.ds(mcb, 16)] = s2v
            return mcb + c2

        return lax.fori_loop(0, NV2, scan_pair, jnp.int32(0))

    def gather_reduce(mc):
        ng = (mc + (K - 1)) // K

        def g_body(g, _):
            start_gather(g, rows0_v, sem_g0)
            wait_rows(sem_g0, rows0_v)
            accum(rows0_v, g, mc)
            return 0

        lax.fori_loop(0, ng, g_body, 0)

    start_dst(0, dst0_v, sem_d0)
    start_dst(1, dst1_v, sem_d1)

    def chunk_pair(c2, _):
        c = 2 * c2

        wait_dst(sem_d0, dst0_v)
        mc0 = process_chunk(c, dst0_v)

        @pl.when(c + 2 < NCHUNK)
        def _():
            start_dst(c + 2, dst0_v, sem_d0)
        gather_reduce(mc0)

        wait_dst(sem_d1, dst1_v)
        mc1 = process_chunk(c + 1, dst1_v)

        @pl.when(c + 3 < NCHUNK)
        def _():
            start_dst(c + 3, dst1_v, sem_d1)
        gather_reduce(mc1)

        return 0

    lax.fori_loop(0, NCHUNK // 2, chunk_pair, 0)

    wbv = wb_v[pl.ds(0, 16)]
    w0 = jnp.broadcast_to(wbv[0], (16,))
    w1 = jnp.broadcast_to(wbv[1], (16,))
    w2 = jnp.broadcast_to(wbv[2], (16,))
    w3 = jnp.broadcast_to(wbv[3], (16,))
    bb = jnp.broadcast_to(wbv[4], (16,))

    def comb(n, _):
        dg = deg_v[n]
        df = dg.astype(jnp.float32)
        hasv = jnp.broadcast_to((dg > 0).astype(jnp.float32), (16,))
        dfv = jnp.broadcast_to(df, (16,))
        invd = jnp.float32(1.0) / jnp.maximum(dfv, jnp.float32(1.0))
        for f in range(NF):
            sl = pl.ds(f * 16, 16)
            s = sum_v[n, sl]
            mn = min_v[n, sl]
            mx = max_v[n, sl]
            val = w0 * s + hasv * (w1 * mn + w2 * mx) + w3 * (s * invd) + bb
            sum_v[n, sl] = val
        return 0

    lax.fori_loop(0, NPT, comb, 0)

    pltpu.sync_copy(sum_v, out_h.at[pl.ds(wid * NPT_PAD, NPT_PAD)])


@jax.jit
def _agg(m, dst, wb):
    mesh = plsc.VectorSubcoreMesh(core_axis_name="c", subcore_axis_name="s",
                                  num_cores=2, num_subcores=16)
    return pl.kernel(
        _body,
        out_type=jax.ShapeDtypeStruct((NW * NPT_PAD, D), jnp.float32),
        mesh=mesh,
        compiler_params=pltpu.CompilerParams(needs_layout_passes=False),
        scratch_types=[
            pltpu.VMEM((NPT_PAD, D), jnp.float32),
            pltpu.VMEM((NPT, D), jnp.float32),
            pltpu.VMEM((NPT, D), jnp.float32),
            pltpu.SMEM((320,), jnp.int32),
            pltpu.VMEM((C,), jnp.int32),
            pltpu.VMEM((C,), jnp.int32),
            pltpu.VMEM((C + 16,), jnp.int32),
            pltpu.VMEM((C + 16,), jnp.int32),
            pltpu.VMEM((K, D), jnp.float32),
            pltpu.VMEM((K, D), jnp.float32),
            pltpu.VMEM((16,), jnp.float32),
            pltpu.SemaphoreType.DMA,
            pltpu.SemaphoreType.DMA,
            pltpu.SemaphoreType.DMA,
            pltpu.SemaphoreType.DMA,
        ],
    )(m, dst, wb)


def kernel(m, edge_index, W, b):
    dst = edge_index[1]
    wb = jnp.concatenate([W.reshape(-1), b,
                          jnp.zeros((11,), jnp.float32)])
    out_pad = _agg(m, dst, wb)
    out = out_pad.reshape(NW, NPT_PAD, D)[:, :NPT].reshape(NW * NPT, D)
    return out[:N_NODES]


# speedup vs baseline: 2.8563x; 1.2661x over previous
import jax
import jax.numpy as jnp
from jax import lax
from jax.experimental import pallas as pl
from jax.experimental.pallas import tpu as pltpu
from jax.experimental.pallas import tpu_sc as plsc

N_NODES = 10000
N_EDGES = 320000
D = 128
NF = D // 16

NW = 32
NPT = 313
NPT_PAD = 320

C = 800
NCHUNK = N_EDGES // C
NV2 = C // 32
K = 16

def _body(m_h, dst_h, wb_h, out_h,
          sum_v, min_v, max_v, deg_v, dst0_v, dst1_v, ids_v, lds_v,
          rows_v, wb_v, sem_d0, sem_d1, sem_g0):
    cid = lax.axis_index("c")
    sid = lax.axis_index("s")
    wid = sid * 2 + cid
    lo = wid * NPT
    hi = jnp.minimum(lo + NPT, N_NODES)

    pltpu.sync_copy(wb_h, wb_v)

    zero16f = jnp.zeros((16,), jnp.float32)
    pinf = jnp.full((16,), jnp.inf, jnp.float32)
    ninf = jnp.full((16,), -jnp.inf, jnp.float32)
    iota16 = lax.iota(jnp.int32, 16)

    def init_node(n, _):
        for f in range(NF):
            sl = pl.ds(f * 16, 16)
            sum_v[n, sl] = zero16f
            min_v[n, sl] = pinf
            max_v[n, sl] = ninf
        return 0
    lax.fori_loop(0, NPT, init_node, 0)

    zero16i = jnp.zeros((16,), jnp.int32)

    def init_deg(i, _):
        deg_v[i] = jnp.int32(0)
        return 0
    lax.fori_loop(0, 320, init_deg, 0)

    def init_ids(i, _):
        ids_v[pl.ds(i * 16, 16)] = zero16i
        return 0
    lax.fori_loop(0, (C + 16) // 16, init_ids, 0)

    def start_dst(c, dbuf, dsem):
        pltpu.async_copy(dst_h.at[pl.ds(c * C, C)], dbuf, dsem)

    def start_gather(g, half):
        pltpu.async_copy(m_h.at[ids_v.at[pl.ds(g * K, K)]],
                         rows_v.at[pl.ds(half * K, K)], sem_g0)

    def wait_dst(sem, buf):
        pltpu.make_async_copy(dst_h.at[pl.ds(0, C)], buf, sem).wait()

    def wait_rows():
        pltpu.make_async_copy(m_h.at[pl.ds(0, K)],
                              rows_v.at[pl.ds(0, K)], sem_g0).wait()

    def accum(rbase, g, mc):
        kk = jnp.minimum(jnp.int32(K), mc - g * K)

        def e_body(j, _):
            ld = lds_v[pl.ds(g * K + j, 16)][0]
            for f in range(NF):
                sl = pl.ds(f * 16, 16)
                r = rows_v[rbase + j, sl]
                sum_v[ld, sl] = sum_v[ld, sl] + r
                min_v[ld, sl] = jnp.minimum(min_v[ld, sl], r)
                max_v[ld, sl] = jnp.maximum(max_v[ld, sl], r)
            deg_v[ld] = deg_v[ld] + 1
            return 0

        lax.fori_loop(0, kk, e_body, 0)

    def process_chunk(c, dst_v):
        base_id = c * C + iota16

        def scan_pair(v2, mc):
            off = v2 * 32
            d1 = dst_v[pl.ds(off, 16)]
            d2 = dst_v[pl.ds(off + 16, 16)]
            m1 = (d1 >= lo) & (d1 < hi)
            m2 = (d2 >= lo) & (d2 < hi)
            k1 = jnp.where(m1, d1 - lo, jnp.int32(0x7FFFFFFF))
            k2 = jnp.where(m2, d2 - lo, jnp.int32(0x7FFFFFFF))
            s1k, s1v = plsc.sort_key_val(k1, base_id + off)
            s2k, s2v = plsc.sort_key_val(k2, base_id + (off + 16))
            c1 = plsc.all_reduce_population_count(m1)[0]
            c2 = plsc.all_reduce_population_count(m2)[0]
            lds_v[pl.ds(mc, 16)] = s1k
            ids_v[pl.ds(mc, 16)] = s1v
            mcb = mc + c1
            lds_v[pl.ds(mcb, 16)] = s2k
            ids_v[pl.ds(mcb, 16)] = s2v
            return mcb + c2

        return lax.fori_loop(0, NV2, scan_pair, jnp.int32(0))

    def gather_reduce(mc):
        ng = (mc + (K - 1)) // K

        @pl.when(ng > 0)
        def _():
            start_gather(0, 0)

        def g_body(g, _):
            wait_rows()

            @pl.when(g + 1 < ng)
            def _():
                start_gather(g + 1, (g + 1) % 2)
            accum((g % 2) * K, g, mc)
            return 0

        lax.fori_loop(0, ng, g_body, 0)

    start_dst(0, dst0_v, sem_d0)
    start_dst(1, dst1_v, sem_d1)

    def chunk_pair(c2, _):
        c = 2 * c2

        wait_dst(sem_d0, dst0_v)
        mc0 = process_chunk(c, dst0_v)

        @pl.when(c + 2 < NCHUNK)
        def _():
            start_dst(c + 2, dst0_v, sem_d0)
        gather_reduce(mc0)

        wait_dst(sem_d1, dst1_v)
        mc1 = process_chunk(c + 1, dst1_v)

        @pl.when(c + 3 < NCHUNK)
        def _():
            start_dst(c + 3, dst1_v, sem_d1)
        gather_reduce(mc1)

        return 0

    lax.fori_loop(0, NCHUNK // 2, chunk_pair, 0)

    wbv = wb_v[pl.ds(0, 16)]
    w0 = jnp.broadcast_to(wbv[0], (16,))
    w1 = jnp.broadcast_to(wbv[1], (16,))
    w2 = jnp.broadcast_to(wbv[2], (16,))
    w3 = jnp.broadcast_to(wbv[3], (16,))
    bb = jnp.broadcast_to(wbv[4], (16,))

    def comb(n, _):
        dg = deg_v[n]
        df = dg.astype(jnp.float32)
        hasv = jnp.broadcast_to((dg > 0).astype(jnp.float32), (16,))
        dfv = jnp.broadcast_to(df, (16,))
        invd = jnp.float32(1.0) / jnp.maximum(dfv, jnp.float32(1.0))
        for f in range(NF):
            sl = pl.ds(f * 16, 16)
            s = sum_v[n, sl]
            mn = min_v[n, sl]
            mx = max_v[n, sl]
            val = w0 * s + hasv * (w1 * mn + w2 * mx) + w3 * (s * invd) + bb
            sum_v[n, sl] = val
        return 0

    lax.fori_loop(0, NPT, comb, 0)

    pltpu.sync_copy(sum_v, out_h.at[pl.ds(wid * NPT_PAD, NPT_PAD)])


@jax.jit
def _agg(m, dst, wb):
    mesh = plsc.VectorSubcoreMesh(core_axis_name="c", subcore_axis_name="s",
                                  num_cores=2, num_subcores=16)
    return pl.kernel(
        _body,
        out_type=jax.ShapeDtypeStruct((NW * NPT_PAD, D), jnp.float32),
        mesh=mesh,
        compiler_params=pltpu.CompilerParams(needs_layout_passes=False),
        scratch_types=[
            pltpu.VMEM((NPT_PAD, D), jnp.float32),
            pltpu.VMEM((NPT, D), jnp.float32),
            pltpu.VMEM((NPT, D), jnp.float32),
            pltpu.SMEM((320,), jnp.int32),
            pltpu.VMEM((C,), jnp.int32),
            pltpu.VMEM((C,), jnp.int32),
            pltpu.VMEM((C + 16,), jnp.int32),
            pltpu.VMEM((C + 16,), jnp.int32),
            pltpu.VMEM((2 * K, D), jnp.float32),
            pltpu.VMEM((16,), jnp.float32),
            pltpu.SemaphoreType.DMA,
            pltpu.SemaphoreType.DMA,
            pltpu.SemaphoreType.DMA,
        ],
    )(m, dst, wb)


def kernel(m, edge_index, W, b):
    dst = edge_index[1]
    wb = jnp.concatenate([W.reshape(-1), b,
                          jnp.zeros((11,), jnp.float32)])
    out_pad = _agg(m, dst, wb)
    out = out_pad.reshape(NW, NPT_PAD, D)[:, :NPT].reshape(NW * NPT, D)
    return out[:N_NODES]


# speedup vs baseline: 2.8821x; 1.0091x over previous
import jax
import jax.numpy as jnp
from jax import lax
from jax.experimental import pallas as pl
from jax.experimental.pallas import tpu as pltpu
from jax.experimental.pallas import tpu_sc as plsc

N_NODES = 10000
N_EDGES = 320000
D = 128
NF = D // 16

NW = 32
NPT = 313
NPT_PAD = 320

C = 800
NCHUNK = N_EDGES // C
NV2 = C // 32
K = 8

def _body(m_h, dst_h, wb_h, out_h,
          sum_v, min_v, max_v, deg_v, dst0_v, dst1_v, ids0_v, lds0_v,
          ids1_v, lds1_v, rows_v, wb_v, sem_d0, sem_d1, sem_g0):
    cid = lax.axis_index("c")
    sid = lax.axis_index("s")
    wid = sid * 2 + cid
    lo = wid * NPT
    hi = jnp.minimum(lo + NPT, N_NODES)

    pltpu.sync_copy(wb_h, wb_v)

    zero16f = jnp.zeros((16,), jnp.float32)
    pinf = jnp.full((16,), jnp.inf, jnp.float32)
    ninf = jnp.full((16,), -jnp.inf, jnp.float32)
    iota16 = lax.iota(jnp.int32, 16)

    def init_node(n, _):
        for f in range(NF):
            sl = pl.ds(f * 16, 16)
            sum_v[n, sl] = zero16f
            min_v[n, sl] = pinf
            max_v[n, sl] = ninf
        return 0
    lax.fori_loop(0, NPT, init_node, 0)

    zero16i = jnp.zeros((16,), jnp.int32)

    def init_deg(i, _):
        deg_v[i] = jnp.int32(0)
        return 0
    lax.fori_loop(0, 320, init_deg, 0)

    def init_ids(i, _):
        ids0_v[pl.ds(i * 16, 16)] = zero16i
        ids1_v[pl.ds(i * 16, 16)] = zero16i
        return 0
    lax.fori_loop(0, (C + 16) // 16, init_ids, 0)

    def start_dst(c, dbuf, dsem):
        pltpu.async_copy(dst_h.at[pl.ds(c * C, C)], dbuf, dsem)

    def start_gather(ids_v, g, half):
        pltpu.async_copy(m_h.at[ids_v.at[pl.ds(g * K, K)]],
                         rows_v.at[pl.ds(half * K, K)], sem_g0)

    def wait_dst(sem, buf):
        pltpu.make_async_copy(dst_h.at[pl.ds(0, C)], buf, sem).wait()

    def wait_rows():
        pltpu.make_async_copy(m_h.at[pl.ds(0, K)],
                              rows_v.at[pl.ds(0, K)], sem_g0).wait()

    def accum(lds_v, half, g, mc):
        kk = jnp.minimum(jnp.int32(K), mc - g * K)
        rbase = half * K

        def e_body(j, _):
            ld = lds_v[pl.ds(g * K + j, 16)][0]
            for f in range(NF):
                sl = pl.ds(f * 16, 16)
                r = rows_v[rbase + j, sl]
                sum_v[ld, sl] = sum_v[ld, sl] + r
                min_v[ld, sl] = jnp.minimum(min_v[ld, sl], r)
                max_v[ld, sl] = jnp.maximum(max_v[ld, sl], r)
            deg_v[ld] = deg_v[ld] + 1
            return 0

        lax.fori_loop(0, kk, e_body, 0)

    def process_chunk(c, dst_v, ids_v, lds_v):
        base_id = c * C + iota16

        def scan_pair(v2, mc):
            off = v2 * 32
            d1 = dst_v[pl.ds(off, 16)]
            d2 = dst_v[pl.ds(off + 16, 16)]
            m1 = (d1 >= lo) & (d1 < hi)
            m2 = (d2 >= lo) & (d2 < hi)
            k1 = jnp.where(m1, d1 - lo, jnp.int32(0x7FFFFFFF))
            k2 = jnp.where(m2, d2 - lo, jnp.int32(0x7FFFFFFF))
            s1k, s1v = plsc.sort_key_val(k1, base_id + off)
            s2k, s2v = plsc.sort_key_val(k2, base_id + (off + 16))
            c1 = plsc.all_reduce_population_count(m1)[0]
            c2 = plsc.all_reduce_population_count(m2)[0]
            lds_v[pl.ds(mc, 16)] = s1k
            ids_v[pl.ds(mc, 16)] = s1v
            mcb = mc + c1
            lds_v[pl.ds(mcb, 16)] = s2k
            ids_v[pl.ds(mcb, 16)] = s2v
            return mcb + c2

        return lax.fori_loop(0, NV2, scan_pair, jnp.int32(0))


    start_dst(0, dst0_v, sem_d0)
    start_dst(1, dst1_v, sem_d1)

    def chunk_pair(c2, h):
        c = 2 * c2

        wait_dst(sem_d0, dst0_v)
        mc0 = process_chunk(c, dst0_v, ids0_v, lds0_v)

        @pl.when(c + 2 < NCHUNK)
        def _():
            start_dst(c + 2, dst0_v, sem_d0)
        ng0 = (mc0 + (K - 1)) // K

        @pl.when(ng0 > 0)
        def _():
            start_gather(ids0_v, 0, h)

        wait_dst(sem_d1, dst1_v)
        mc1 = process_chunk(c + 1, dst1_v, ids1_v, lds1_v)

        @pl.when(c + 3 < NCHUNK)
        def _():
            start_dst(c + 3, dst1_v, sem_d1)
        ng1 = (mc1 + (K - 1)) // K

        def body0(g, _):
            wait_rows()
            hg = (h + g) % 2

            @pl.when(g + 1 < ng0)
            def _():
                start_gather(ids0_v, g + 1, (hg + 1) % 2)

            @pl.when((g + 1 == ng0) & (ng1 > 0))
            def _():
                start_gather(ids1_v, 0, (hg + 1) % 2)
            accum(lds0_v, hg, g, mc0)
            return 0

        lax.fori_loop(0, ng0, body0, 0)

        @pl.when((ng0 == 0) & (ng1 > 0))
        def _():
            start_gather(ids1_v, 0, h)

        def body1(g, _):
            wait_rows()
            hg = (h + ng0 + g) % 2

            @pl.when(g + 1 < ng1)
            def _():
                start_gather(ids1_v, g + 1, (hg + 1) % 2)
            accum(lds1_v, hg, g, mc1)
            return 0

        lax.fori_loop(0, ng1, body1, 0)

        return (h + ng0 + ng1) % 2

    lax.fori_loop(0, NCHUNK // 2, chunk_pair, jnp.int32(0))

    wbv = wb_v[pl.ds(0, 16)]
    w0 = jnp.broadcast_to(wbv[0], (16,))
    w1 = jnp.broadcast_to(wbv[1], (16,))
    w2 = jnp.broadcast_to(wbv[2], (16,))
    w3 = jnp.broadcast_to(wbv[3], (16,))
    bb = jnp.broadcast_to(wbv[4], (16,))

    def comb(n, _):
        dg = deg_v[n]
        df = dg.astype(jnp.float32)
        hasv = jnp.broadcast_to((dg > 0).astype(jnp.float32), (16,))
        dfv = jnp.broadcast_to(df, (16,))
        invd = jnp.float32(1.0) / jnp.maximum(dfv, jnp.float32(1.0))
        for f in range(NF):
            sl = pl.ds(f * 16, 16)
            s = sum_v[n, sl]
            mn = min_v[n, sl]
            mx = max_v[n, sl]
            val = w0 * s + hasv * (w1 * mn + w2 * mx) + w3 * (s * invd) + bb
            sum_v[n, sl] = val
        return 0

    lax.fori_loop(0, NPT, comb, 0)

    pltpu.sync_copy(sum_v, out_h.at[pl.ds(wid * NPT_PAD, NPT_PAD)])


@jax.jit
def _agg(m, dst, wb):
    mesh = plsc.VectorSubcoreMesh(core_axis_name="c", subcore_axis_name="s",
                                  num_cores=2, num_subcores=16)
    return pl.kernel(
        _body,
        out_type=jax.ShapeDtypeStruct((NW * NPT_PAD, D), jnp.float32),
        mesh=mesh,
        compiler_params=pltpu.CompilerParams(needs_layout_passes=False),
        scratch_types=[
            pltpu.VMEM((NPT_PAD, D), jnp.float32),
            pltpu.VMEM((NPT, D), jnp.float32),
            pltpu.VMEM((NPT, D), jnp.float32),
            pltpu.SMEM((320,), jnp.int32),
            pltpu.VMEM((C,), jnp.int32),
            pltpu.VMEM((C,), jnp.int32),
            pltpu.VMEM((C + 16,), jnp.int32),
            pltpu.VMEM((C + 16,), jnp.int32),
            pltpu.VMEM((C + 16,), jnp.int32),
            pltpu.VMEM((C + 16,), jnp.int32),
            pltpu.VMEM((2 * K, D), jnp.float32),
            pltpu.VMEM((16,), jnp.float32),
            pltpu.SemaphoreType.DMA,
            pltpu.SemaphoreType.DMA,
            pltpu.SemaphoreType.DMA,
        ],
    )(m, dst, wb)


def kernel(m, edge_index, W, b):
    dst = edge_index[1]
    wb = jnp.concatenate([W.reshape(-1), b,
                          jnp.zeros((11,), jnp.float32)])
    out_pad = _agg(m, dst, wb)
    out = out_pad.reshape(NW, NPT_PAD, D)[:, :NPT].reshape(NW * NPT, D)
    return out[:N_NODES]


# speedup vs baseline: 2.9072x; 1.0087x over previous
import jax
import jax.numpy as jnp
from jax import lax
from jax.experimental import pallas as pl
from jax.experimental.pallas import tpu as pltpu
from jax.experimental.pallas import tpu_sc as plsc

N_NODES = 10000
N_EDGES = 320000
D = 128
NF = D // 16

NW = 32
NPT = 313
NPT_PAD = 320

C = 800
NCHUNK = N_EDGES // C
NV2 = C // 32
K = 8

def _body(m_h, dst_h, wb_h, out_h,
          sum_v, min_v, max_v, deg_v, dst0_v, dst1_v, ids0_v, lds0_v,
          ids1_v, lds1_v, rows_v, wb_v, sem_d0, sem_d1, sem_g0):
    cid = lax.axis_index("c")
    sid = lax.axis_index("s")
    wid = sid * 2 + cid
    lo = wid * NPT
    hi = jnp.minimum(lo + NPT, N_NODES)

    pltpu.sync_copy(wb_h, wb_v)

    zero16f = jnp.zeros((16,), jnp.float32)
    pinf = jnp.full((16,), jnp.inf, jnp.float32)
    ninf = jnp.full((16,), -jnp.inf, jnp.float32)
    iota16 = lax.iota(jnp.int32, 16)

    def init_node(n, _):
        for f in range(NF):
            sl = pl.ds(f * 16, 16)
            sum_v[n, sl] = zero16f
            min_v[n, sl] = pinf
            max_v[n, sl] = ninf
        return 0
    lax.fori_loop(0, NPT, init_node, 0)

    zero16i = jnp.zeros((16,), jnp.int32)

    def init_deg(i, _):
        deg_v[i] = jnp.int32(0)
        return 0
    lax.fori_loop(0, 320, init_deg, 0)

    def init_ids(i, _):
        ids0_v[pl.ds(i * 16, 16)] = zero16i
        ids1_v[pl.ds(i * 16, 16)] = zero16i
        return 0
    lax.fori_loop(0, (C + 16) // 16, init_ids, 0)

    def start_dst(c, dbuf, dsem):
        pltpu.async_copy(dst_h.at[pl.ds(c * C, C)], dbuf, dsem)

    def start_gather(ids_v, g, half):
        pltpu.async_copy(m_h.at[ids_v.at[pl.ds(g * K, K)]],
                         rows_v.at[pl.ds(half * K, K)], sem_g0)

    def wait_dst(sem, buf):
        pltpu.make_async_copy(dst_h.at[pl.ds(0, C)], buf, sem).wait()

    def wait_rows():
        pltpu.make_async_copy(m_h.at[pl.ds(0, K)],
                              rows_v.at[pl.ds(0, K)], sem_g0).wait()

    def accum(lds_v, half, g, mc):
        kk = jnp.minimum(jnp.int32(K), mc - g * K)
        rbase = half * K
        ldv = lds_v[pl.ds(g * K, 16)]
        for j in range(K):
            @pl.when(j < kk)
            def _(j=j):
                ld = ldv[j]
                for f in range(NF):
                    sl = pl.ds(f * 16, 16)
                    r = rows_v[rbase + j, sl]
                    sum_v[ld, sl] = sum_v[ld, sl] + r
                    min_v[ld, sl] = jnp.minimum(min_v[ld, sl], r)
                    max_v[ld, sl] = jnp.maximum(max_v[ld, sl], r)
                deg_v[ld] = deg_v[ld] + 1

    def process_chunk(c, dst_v, ids_v, lds_v):
        base_id = c * C + iota16

        def scan_block(off, mc, nvec):
            sks, svs, cs = [], [], []
            for u in range(nvec):
                d = dst_v[pl.ds(off + u * 16, 16)]
                mm = (d >= lo) & (d < hi)
                kk = jnp.where(mm, d - lo, jnp.int32(0x7FFFFFFF))
                sk, sv = plsc.sort_key_val(kk, base_id + (off + u * 16))
                sks.append(sk)
                svs.append(sv)
                cs.append(plsc.all_reduce_population_count(mm)[0])
            for u in range(nvec):
                lds_v[pl.ds(mc, 16)] = sks[u]
                ids_v[pl.ds(mc, 16)] = svs[u]
                mc = mc + cs[u]
            return mc

        def scan_quad(q, mc):
            return scan_block(q * 64, mc, 4)

        mc = lax.fori_loop(0, (C - 32) // 64, scan_quad, jnp.int32(0))
        return scan_block(C - 32, mc, 2)


    start_dst(0, dst0_v, sem_d0)
    start_dst(1, dst1_v, sem_d1)

    def chunk_pair(c2, h):
        c = 2 * c2

        wait_dst(sem_d0, dst0_v)
        mc0 = process_chunk(c, dst0_v, ids0_v, lds0_v)

        @pl.when(c + 2 < NCHUNK)
        def _():
            start_dst(c + 2, dst0_v, sem_d0)
        ng0 = (mc0 + (K - 1)) // K

        @pl.when(ng0 > 0)
        def _():
            start_gather(ids0_v, 0, h)

        wait_dst(sem_d1, dst1_v)
        mc1 = process_chunk(c + 1, dst1_v, ids1_v, lds1_v)

        @pl.when(c + 3 < NCHUNK)
        def _():
            start_dst(c + 3, dst1_v, sem_d1)
        ng1 = (mc1 + (K - 1)) // K

        def body0(g, _):
            wait_rows()
            hg = (h + g) % 2

            @pl.when(g + 1 < ng0)
            def _():
                start_gather(ids0_v, g + 1, (hg + 1) % 2)

            @pl.when((g + 1 == ng0) & (ng1 > 0))
            def _():
                start_gather(ids1_v, 0, (hg + 1) % 2)
            accum(lds0_v, hg, g, mc0)
            return 0

        lax.fori_loop(0, ng0, body0, 0)

        @pl.when((ng0 == 0) & (ng1 > 0))
        def _():
            start_gather(ids1_v, 0, h)

        def body1(g, _):
            wait_rows()
            hg = (h + ng0 + g) % 2

            @pl.when(g + 1 < ng1)
            def _():
                start_gather(ids1_v, g + 1, (hg + 1) % 2)
            accum(lds1_v, hg, g, mc1)
            return 0

        lax.fori_loop(0, ng1, body1, 0)

        return (h + ng0 + ng1) % 2

    lax.fori_loop(0, NCHUNK // 2, chunk_pair, jnp.int32(0))

    wbv = wb_v[pl.ds(0, 16)]
    w0 = jnp.broadcast_to(wbv[0], (16,))
    w1 = jnp.broadcast_to(wbv[1], (16,))
    w2 = jnp.broadcast_to(wbv[2], (16,))
    w3 = jnp.broadcast_to(wbv[3], (16,))
    bb = jnp.broadcast_to(wbv[4], (16,))

    def comb(n, _):
        dg = deg_v[n]
        df = dg.astype(jnp.float32)
        hasv = jnp.broadcast_to((dg > 0).astype(jnp.float32), (16,))
        dfv = jnp.broadcast_to(df, (16,))
        invd = jnp.float32(1.0) / jnp.maximum(dfv, jnp.float32(1.0))
        for f in range(NF):
            sl = pl.ds(f * 16, 16)
            s = sum_v[n, sl]
            mn = min_v[n, sl]
            mx = max_v[n, sl]
            val = w0 * s + hasv * (w1 * mn + w2 * mx) + w3 * (s * invd) + bb
            sum_v[n, sl] = val
        return 0

    lax.fori_loop(0, NPT, comb, 0)

    pltpu.sync_copy(sum_v, out_h.at[pl.ds(wid * NPT_PAD, NPT_PAD)])


@jax.jit
def _agg(m, dst, wb):
    mesh = plsc.VectorSubcoreMesh(core_axis_name="c", subcore_axis_name="s",
                                  num_cores=2, num_subcores=16)
    return pl.kernel(
        _body,
        out_type=jax.ShapeDtypeStruct((NW * NPT_PAD, D), jnp.float32),
        mesh=mesh,
        compiler_params=pltpu.CompilerParams(needs_layout_passes=False),
        scratch_types=[
            pltpu.VMEM((NPT_PAD, D), jnp.float32),
            pltpu.VMEM((NPT, D), jnp.float32),
            pltpu.VMEM((NPT, D), jnp.float32),
            pltpu.SMEM((320,), jnp.int32),
            pltpu.VMEM((C,), jnp.int32),
            pltpu.VMEM((C,), jnp.int32),
            pltpu.VMEM((C + 16,), jnp.int32),
            pltpu.VMEM((C + 16,), jnp.int32),
            pltpu.VMEM((C + 16,), jnp.int32),
            pltpu.VMEM((C + 16,), jnp.int32),
            pltpu.VMEM((2 * K, D), jnp.float32),
            pltpu.VMEM((16,), jnp.float32),
            pltpu.SemaphoreType.DMA,
            pltpu.SemaphoreType.DMA,
            pltpu.SemaphoreType.DMA,
        ],
    )(m, dst, wb)


def kernel(m, edge_index, W, b):
    dst = edge_index[1]
    wb = jnp.concatenate([W.reshape(-1), b,
                          jnp.zeros((11,), jnp.float32)])
    out_pad = _agg(m, dst, wb)
    out = out_pad.reshape(NW, NPT_PAD, D)[:, :NPT].reshape(NW * NPT, D)
    return out[:N_NODES]
